# baseline (speedup 1.0000x reference)
import functools

import jax
import jax.numpy as jnp
from jax import lax
from jax.experimental import pallas as pl
from jax.experimental.pallas import tpu as pltpu
from jax.experimental.pallas import tpu_sc as plsc

N_NODES = 10000
CH = 128
BM = 1000

NC = 2
NS = 16
KB = 64
IW = 64
SUP = 16
NBUF = 4
ROWS_PER_TILE = 624
_TAIL = N_NODES - NS * ROWS_PER_TILE


def _dense_body(y_ref, w_ref, b_ref, o_ref):
    v = lax.dot_general(y_ref[0], w_ref[0], (((1,), (1,)), ((), ())),
                        preferred_element_type=jnp.float32) + b_ref[0]
    o_ref[0] = jnp.where(v > 0, v, jnp.exp(v) - 1.0)


def _dense(y, W, b):
    win = y.shape[0]
    nblk = N_NODES // BM
    return pl.pallas_call(
        _dense_body,
        grid=(win, nblk),
        in_specs=[
            pl.BlockSpec((1, BM, CH), lambda w, i: (w, i, 0)),
            pl.BlockSpec((1, CH, CH), lambda w, i: (w, 0, 0)),
            pl.BlockSpec((1, 1, CH), lambda w, i: (w, 0, 0)),
        ],
        out_specs=pl.BlockSpec((1, BM, CH), lambda w, i: (w, i, 0)),
        out_shape=jax.ShapeDtypeStruct((win, N_NODES, CH), jnp.float32),
    )(y, W, b[:, None, :])


def _sc_scatter_body(nchunk, src_hbm, dst_hbm, ew_hbm, x_hbm, zeros_hbm,
                     out_hbm, src_v, dst_v, ew_v, rows_v,
                     sem_g0, sem_g1, sem_g2, sem_g3,
                     sem_s0, sem_s1, sem_s2, sem_s3, acc_sh):
    c = lax.axis_index("c")
    s = lax.axis_index("s")
    row0 = s * ROWS_PER_TILE
    nsup = nchunk // SUP
    tile_row0 = (c * NS + s) * nchunk

    sem_g = (sem_g0, sem_g1, sem_g2, sem_g3)
    sem_s = (sem_s0, sem_s1, sem_s2, sem_s3)

    pltpu.sync_copy(zeros_hbm.at[pl.ds(row0, ROWS_PER_TILE)],
                    acc_sh.at[pl.ds(row0, ROWS_PER_TILE)])

    @pl.when(s == NS - 1)
    def _zero_tail():
        pltpu.sync_copy(zeros_hbm.at[pl.ds(NS * ROWS_PER_TILE, _TAIL)],
                        acc_sh.at[pl.ds(NS * ROWS_PER_TILE, _TAIL)])

    def stage(k, slot):
        row_off = tile_row0 + k * SUP
        pltpu.sync_copy(src_hbm.at[pl.ds(row_off, SUP)],
                        src_v.at[pl.ds(slot * SUP, SUP)])
        pltpu.sync_copy(dst_hbm.at[pl.ds(row_off, SUP)],
                        dst_v.at[pl.ds(slot * SUP, SUP)])
        pltpu.sync_copy(ew_hbm.at[pl.ds(row_off, SUP)],
                        ew_v.at[pl.ds(slot * SUP, SUP)])

    def gather_desc(slot, h, b):
        return pltpu.make_async_copy(
            x_hbm.at[src_v.at[slot * SUP + h]], rows_v.at[b], sem_g[b])

    def scatter_desc(slot, h, b):
        return pltpu.make_async_copy(
            rows_v.at[b], acc_sh.at[dst_v.at[slot * SUP + h]], sem_s[b])

    def scale(slot, h, b):
        def scale16(eg, c2):
            e0 = eg * 16
            ew16 = ew_v[slot * SUP + h, pl.ds(e0, 16)]
            for l in range(16):
                wv = jnp.broadcast_to(ew16[l], (16,))
                for jj in range(CH // 16):
                    sl = pl.ds(jj * 16, 16)
                    rows_v[b, e0 + l, sl] = rows_v[b, e0 + l, sl] * wv
            return c2

        lax.fori_loop(0, KB // 16, scale16, 0)

    stage(0, 0)
    plsc.subcore_barrier()
    gather_desc(0, 0, 0).start()

    def superchunk(k, carry):
        slot = k & 1
        nslot = 1 - slot

        for h in range(SUP):
            b = h % NBUF
            gather_desc(slot, h, b).wait()

            if h >= 2:
                scatter_desc(slot, h - 2, (b + 2) % NBUF).wait()
            else:
                @pl.when(k > 0)
                def _drain():
                    scatter_desc(nslot, SUP - 2 + h, (b + 2) % NBUF).wait()

            if h < SUP - 1:
                gather_desc(slot, h + 1, (b + 1) % NBUF).start()
            else:
                @pl.when(k + 1 < nsup)
                def _pref():
                    gather_desc(nslot, 0, (b + 1) % NBUF).start()

            scale(slot, h, b)

            scatter_desc(slot, h, b).start(add=True)

            if h == 2:
                @pl.when(k + 1 < nsup)
                def _stage_next():
                    stage(k + 1, nslot)

        return carry

    lax.fori_loop(0, nsup, superchunk, 0)

    last_slot = (nsup - 1) & 1
    scatter_desc(last_slot, SUP - 2, (SUP - 2) % NBUF).wait()
    scatter_desc(last_slot, SUP - 1, (SUP - 1) % NBUF).wait()

    plsc.subcore_barrier()

    pltpu.sync_copy(acc_sh.at[pl.ds(row0, ROWS_PER_TILE)],
                    out_hbm.at[c, pl.ds(row0, ROWS_PER_TILE)])

    @pl.when(s == NS - 1)
    def _out_tail():
        pltpu.sync_copy(acc_sh.at[pl.ds(NS * ROWS_PER_TILE, _TAIL)],
                        out_hbm.at[c, pl.ds(NS * ROWS_PER_TILE, _TAIL)])


def _sc_scatter(x_flat, src2d, dst2d, ew2d, zeros, nchunk):
    win = NC
    mesh = plsc.VectorSubcoreMesh(core_axis_name="c", subcore_axis_name="s")
    f = pl.kernel(
        functools.partial(_sc_scatter_body, nchunk),
        out_type=jax.ShapeDtypeStruct((win, N_NODES, CH), jnp.float32),
        mesh=mesh,
        scratch_types=[
            pltpu.VMEM((2 * SUP, IW), jnp.int32),
            pltpu.VMEM((2 * SUP, IW), jnp.int32),
            pltpu.VMEM((2 * SUP, IW), jnp.float32),
            pltpu.VMEM((NBUF, KB, CH), jnp.float32),
            pltpu.SemaphoreType.DMA,
            pltpu.SemaphoreType.DMA,
            pltpu.SemaphoreType.DMA,
            pltpu.SemaphoreType.DMA,
            pltpu.SemaphoreType.DMA,
            pltpu.SemaphoreType.DMA,
            pltpu.SemaphoreType.DMA,
            pltpu.SemaphoreType.DMA,
            pltpu.VMEM_SHARED((N_NODES, CH), jnp.float32),
        ],
    )
    return f(src2d, dst2d, ew2d, x_flat, zeros)


def kernel(x_list, A_list, E_list, W, b):
    win, n_edges = E_list.shape
    hops = W.shape[0]

    nchunk = -(-n_edges // (NS * KB))
    nchunk = -(-nchunk // SUP) * SUP
    per_tile = nchunk * KB
    e_pad = per_tile * NS
    pad = e_pad - n_edges

    src = A_list[:, 0, :].astype(jnp.int32)
    dst = A_list[:, 1, :].astype(jnp.int32)
    pad_idx = (jnp.arange(pad, dtype=jnp.int32) * 8) % N_NODES
    src = jnp.concatenate([src, jnp.tile(pad_idx[None], (win, 1))], axis=1)
    dst = jnp.concatenate([dst, jnp.tile(pad_idx[None], (win, 1))], axis=1)
    ew = jnp.concatenate(
        [E_list, jnp.zeros((win, pad), jnp.float32)], axis=1)

    src = src + (jnp.arange(win, dtype=jnp.int32) * N_NODES)[:, None]

    nrow = win * NS * nchunk
    src2d = src.reshape(nrow, KB)
    dst2d = dst.reshape(nrow, KB)
    ew2d = ew.reshape(nrow, KB)
    zeros = jnp.zeros((N_NODES, CH), jnp.float32)

    x = x_list
    for j in range(hops):
        y = _sc_scatter(x.reshape(win * N_NODES, CH), src2d, dst2d, ew2d,
                        zeros, nchunk)
        x = _dense(y, W[j], b[j])
    return x


# speedup vs baseline: 1.3234x; 1.3234x over previous
import functools

import jax
import jax.numpy as jnp
from jax import lax
from jax.experimental import pallas as pl
from jax.experimental.pallas import tpu as pltpu
from jax.experimental.pallas import tpu_sc as plsc

N_NODES = 10000
CH = 128
BM = 1000

NC = 2
NS = 16
KB = 128
SUP = 16
ROWS_PER_TILE = 624
_TAIL = N_NODES - NS * ROWS_PER_TILE


def _dense_body(y_ref, w_ref, b_ref, o_ref):
    v = lax.dot_general(y_ref[0], w_ref[0], (((1,), (1,)), ((), ())),
                        preferred_element_type=jnp.float32) + b_ref[0]
    o_ref[0] = jnp.where(v > 0, v, jnp.exp(v) - 1.0)


def _dense(y, W, b):
    win = y.shape[0]
    nblk = N_NODES // BM
    return pl.pallas_call(
        _dense_body,
        grid=(win, nblk),
        in_specs=[
            pl.BlockSpec((1, BM, CH), lambda w, i: (w, i, 0)),
            pl.BlockSpec((1, CH, CH), lambda w, i: (w, 0, 0)),
            pl.BlockSpec((1, 1, CH), lambda w, i: (w, 0, 0)),
        ],
        out_specs=pl.BlockSpec((1, BM, CH), lambda w, i: (w, i, 0)),
        out_shape=jax.ShapeDtypeStruct((win, N_NODES, CH), jnp.float32),
    )(y, W, b[:, None, :])


def _sc_scatter_body(nchunk, src_hbm, dst_hbm, ew_hbm, x_hbm, zeros_hbm,
                     out_hbm, src_v, dst_v, ew_v, rows_v,
                     sem_g0, sem_g1, sem_s0, sem_s1, acc_sh):
    c = lax.axis_index("c")
    s = lax.axis_index("s")
    row0 = s * ROWS_PER_TILE
    nsup = nchunk // SUP
    tile_row0 = (c * NS + s) * nchunk

    sem_g = (sem_g0, sem_g1)
    sem_s = (sem_s0, sem_s1)

    pltpu.sync_copy(zeros_hbm.at[pl.ds(row0, ROWS_PER_TILE)],
                    acc_sh.at[pl.ds(row0, ROWS_PER_TILE)])

    @pl.when(s == NS - 1)
    def _zero_tail():
        pltpu.sync_copy(zeros_hbm.at[pl.ds(NS * ROWS_PER_TILE, _TAIL)],
                        acc_sh.at[pl.ds(NS * ROWS_PER_TILE, _TAIL)])

    def stage(k, slot):
        row_off = tile_row0 + k * SUP
        pltpu.sync_copy(src_hbm.at[pl.ds(row_off, SUP)],
                        src_v.at[pl.ds(slot * SUP, SUP)])
        pltpu.sync_copy(dst_hbm.at[pl.ds(row_off, SUP)],
                        dst_v.at[pl.ds(slot * SUP, SUP)])
        pltpu.sync_copy(ew_hbm.at[pl.ds(row_off * KB, SUP * KB)],
                        ew_v.at[pl.ds(slot * SUP * KB, SUP * KB)])

    def gather_desc(slot, h, b):
        return pltpu.make_async_copy(
            x_hbm.at[src_v.at[slot * SUP + h]], rows_v.at[b], sem_g[b])

    def scatter_desc(slot, h, b):
        return pltpu.make_async_copy(
            rows_v.at[b], acc_sh.at[dst_v.at[slot * SUP + h]], sem_s[b])

    def scale(slot, h, b):
        def scale16(eg, c2):
            e0 = eg * 16
            ew16 = ew_v[pl.ds(slot * SUP * KB + h * KB + e0, 16)]
            for l in range(16):
                wv = jnp.broadcast_to(ew16[l], (16,))
                for jj in range(CH // 16):
                    sl = pl.ds(jj * 16, 16)
                    rows_v[b, e0 + l, sl] = rows_v[b, e0 + l, sl] * wv
            return c2

        lax.fori_loop(0, KB // 16, scale16, 0)

    stage(0, 0)
    plsc.subcore_barrier()
    gather_desc(0, 0, 0).start()

    def superchunk(k, carry):
        slot = k & 1
        nslot = 1 - slot

        for h in range(SUP):
            b = h & 1
            gather_desc(slot, h, b).wait()
            if h < SUP - 1:
                gather_desc(slot, h + 1, 1 - b).start()
            else:
                @pl.when(k + 1 < nsup)
                def _pref():
                    gather_desc(nslot, 0, 1 - b).start()

            scale(slot, h, b)
            pltpu.sync_copy(rows_v.at[b],
                            acc_sh.at[dst_v.at[slot * SUP + h]], add=True)

            if h == 1:
                @pl.when(k + 1 < nsup)
                def _stage_next():
                    stage(k + 1, nslot)

        return carry

    lax.fori_loop(0, nsup, superchunk, 0)

    plsc.subcore_barrier()

    pltpu.sync_copy(acc_sh.at[pl.ds(row0, ROWS_PER_TILE)],
                    out_hbm.at[c, pl.ds(row0, ROWS_PER_TILE)])

    @pl.when(s == NS - 1)
    def _out_tail():
        pltpu.sync_copy(acc_sh.at[pl.ds(NS * ROWS_PER_TILE, _TAIL)],
                        out_hbm.at[c, pl.ds(NS * ROWS_PER_TILE, _TAIL)])


def _sc_scatter(x_flat, src2d, dst2d, ew_flat, zeros, nchunk):
    win = NC
    mesh = plsc.VectorSubcoreMesh(core_axis_name="c", subcore_axis_name="s")
    f = pl.kernel(
        functools.partial(_sc_scatter_body, nchunk),
        out_type=jax.ShapeDtypeStruct((win, N_NODES, CH), jnp.float32),
        mesh=mesh,
        scratch_types=[
            pltpu.VMEM((2 * SUP, KB), jnp.int32),
            pltpu.VMEM((2 * SUP, KB), jnp.int32),
            pltpu.VMEM((2 * SUP * KB,), jnp.float32),
            pltpu.VMEM((2, KB, CH), jnp.float32),
            pltpu.SemaphoreType.DMA,
            pltpu.SemaphoreType.DMA,
            pltpu.SemaphoreType.DMA,
            pltpu.SemaphoreType.DMA,
            pltpu.VMEM_SHARED((N_NODES, CH), jnp.float32),
        ],
    )
    return f(src2d, dst2d, ew_flat, x_flat, zeros)


def kernel(x_list, A_list, E_list, W, b):
    win, n_edges = E_list.shape
    hops = W.shape[0]

    nchunk = -(-n_edges // (NS * KB))
    nchunk = -(-nchunk // SUP) * SUP
    per_tile = nchunk * KB
    e_pad = per_tile * NS
    pad = e_pad - n_edges

    src = A_list[:, 0, :].astype(jnp.int32)
    dst = A_list[:, 1, :].astype(jnp.int32)
    pad_idx = (jnp.arange(pad, dtype=jnp.int32) * 8) % N_NODES
    src = jnp.concatenate([src, jnp.tile(pad_idx[None], (win, 1))], axis=1)
    dst = jnp.concatenate([dst, jnp.tile(pad_idx[None], (win, 1))], axis=1)
    ew = jnp.concatenate(
        [E_list, jnp.zeros((win, pad), jnp.float32)], axis=1)

    src = src + (jnp.arange(win, dtype=jnp.int32) * N_NODES)[:, None]

    src2d = src.reshape(win * NS * nchunk, KB)
    dst2d = dst.reshape(win * NS * nchunk, KB)
    ew_flat = ew.reshape(-1)
    zeros = jnp.zeros((N_NODES, CH), jnp.float32)

    x = x_list
    for j in range(hops):
        y = _sc_scatter(x.reshape(win * N_NODES, CH), src2d, dst2d, ew_flat,
                        zeros, nchunk)
        x = _dense(y, W[j], b[j])
    return x
